# baseline (speedup 1.0000x reference)
import functools

import jax
import jax.numpy as jnp
from jax import lax
from jax.experimental import pallas as pl
from jax.experimental.pallas import tpu as pltpu
from jax.experimental.pallas import tpu_sc as plsc

V = 1000000
D = 64
B = 425984
NC = 2
NS = 16
NW = NC * NS
BPW = B // NW
CHUNK = 1024
NCHUNK = BPW // CHUNK

_mesh = plsc.VectorSubcoreMesh(core_axis_name="c", subcore_axis_name="s")


@functools.partial(
    pl.kernel,
    mesh=_mesh,
    out_type=jax.ShapeDtypeStruct((B, D), jnp.float32),
    scratch_types=[
        pltpu.VMEM((CHUNK,), jnp.int32),
        pltpu.VMEM((CHUNK, D), jnp.float32),
        pltpu.SemaphoreType.DMA,
    ],
    compiler_params=pltpu.CompilerParams(use_tc_tiling_on_sc=False),
)
def _gather(x_hbm, y_hbm, out_hbm, idx_v, rows_v, sem):
    wid = lax.axis_index("s") * NC + lax.axis_index("c")
    base = wid * BPW
    for i in range(NCHUNK):
        off = base + i * CHUNK
        pltpu.sync_copy(y_hbm.at[pl.ds(off, CHUNK)], idx_v)
        pltpu.async_copy(x_hbm.at[idx_v], rows_v, sem).wait()
        pltpu.sync_copy(rows_v, out_hbm.at[pl.ds(off, CHUNK)])


def kernel(x, y):
    return _gather(x, y)


# speedup vs baseline: 1.0065x; 1.0065x over previous
import functools

import jax
import jax.numpy as jnp
from jax import lax
from jax.experimental import pallas as pl
from jax.experimental.pallas import tpu as pltpu
from jax.experimental.pallas import tpu_sc as plsc

V = 1000000
D = 64
B = 425984
NC = 2
NS = 16
NW = NC * NS
BPW = B // NW
CHUNK = 832
NCHUNK = BPW // CHUNK

_mesh = plsc.VectorSubcoreMesh(core_axis_name="c", subcore_axis_name="s")


@functools.partial(
    pl.kernel,
    mesh=_mesh,
    out_type=jax.ShapeDtypeStruct((B, D), jnp.float32),
    scratch_types=[
        pltpu.VMEM((2, CHUNK), jnp.int32),
        pltpu.VMEM((2, CHUNK, D), jnp.float32),
        pltpu.SemaphoreType.DMA,
        pltpu.SemaphoreType.DMA,
        pltpu.SemaphoreType.DMA,
        pltpu.SemaphoreType.DMA,
        pltpu.SemaphoreType.DMA,
    ],
    compiler_params=pltpu.CompilerParams(use_tc_tiling_on_sc=False),
)
def _gather(x_hbm, y_hbm, out_hbm, idx_v, rows_v,
            sem_i0, sem_i1, sem_g, sem_o0, sem_o1):
    wid = lax.axis_index("s") * NC + lax.axis_index("c")
    base = wid * BPW
    sem_i = (sem_i0, sem_i1)
    sem_o = (sem_o0, sem_o1)

    idx_cp = [None] * NCHUNK
    out_cp = [None] * NCHUNK

    def start_idx(i):
        idx_cp[i] = pltpu.async_copy(
            y_hbm.at[pl.ds(base + i * CHUNK, CHUNK)], idx_v.at[i % 2],
            sem_i[i % 2])

    start_idx(0)
    for i in range(NCHUNK):
        b = i % 2
        if i + 1 < NCHUNK:
            start_idx(i + 1)
        if i >= 2:
            out_cp[i - 2].wait()
        idx_cp[i].wait()
        pltpu.async_copy(x_hbm.at[idx_v.at[b]], rows_v.at[b], sem_g).wait()
        out_cp[i] = pltpu.async_copy(
            rows_v.at[b], out_hbm.at[pl.ds(base + i * CHUNK, CHUNK)],
            sem_o[b])
    out_cp[NCHUNK - 2].wait()
    out_cp[NCHUNK - 1].wait()


def kernel(x, y):
    return _gather(x, y)


# speedup vs baseline: 1.0086x; 1.0022x over previous
import functools

import jax
import jax.numpy as jnp
from jax import lax
from jax.experimental import pallas as pl
from jax.experimental.pallas import tpu as pltpu
from jax.experimental.pallas import tpu_sc as plsc

V = 1000000
D = 64
B = 425984
NC = 2
NS = 16
NW = NC * NS
BPW = B // NW
CHUNK = 512
NCHUNK = BPW // CHUNK
NBUF = 3
NIDX = 4
LAG = 2

_mesh = plsc.VectorSubcoreMesh(core_axis_name="c", subcore_axis_name="s")


@functools.partial(
    pl.kernel,
    mesh=_mesh,
    out_type=jax.ShapeDtypeStruct((B, D), jnp.float32),
    scratch_types=[
        pltpu.VMEM((NIDX, CHUNK), jnp.int32),
        pltpu.VMEM((NBUF, CHUNK, D), jnp.float32),
        [pltpu.SemaphoreType.DMA] * NIDX,
        [pltpu.SemaphoreType.DMA] * NBUF,
        [pltpu.SemaphoreType.DMA] * NBUF,
    ],
    compiler_params=pltpu.CompilerParams(use_tc_tiling_on_sc=False),
)
def _gather(x_hbm, y_hbm, out_hbm, idx_v, rows_v, sem_i, sem_g, sem_o):
    wid = lax.axis_index("s") * NC + lax.axis_index("c")
    base = wid * BPW

    idx_cp = [None] * NCHUNK
    g_cp = [None] * NCHUNK
    out_cp = [None] * NCHUNK

    def start_idx(i):
        idx_cp[i] = pltpu.async_copy(
            y_hbm.at[pl.ds(base + i * CHUNK, CHUNK)], idx_v.at[i % NIDX],
            sem_i[i % NIDX])

    for i in range(NIDX):
        start_idx(i)

    for i in range(NCHUNK + LAG):
        if i < NCHUNK:
            b = i % NBUF
            if i >= NBUF:
                out_cp[i - NBUF].wait()
            idx_cp[i].wait()
            g_cp[i] = pltpu.async_copy(
                x_hbm.at[idx_v.at[i % NIDX]], rows_v.at[b], sem_g[b])
        j = i - LAG
        if 0 <= j < NCHUNK:
            g_cp[j].wait()
            out_cp[j] = pltpu.async_copy(
                rows_v.at[j % NBUF],
                out_hbm.at[pl.ds(base + j * CHUNK, CHUNK)],
                sem_o[j % NBUF])
            if j + NIDX < NCHUNK:
                start_idx(j + NIDX)


def kernel(x, y):
    return _gather(x, y)
